# baseline (speedup 1.0000x reference)
import functools

import jax
import jax.numpy as jnp
from jax import lax
from jax.experimental import pallas as pl
from jax.experimental.pallas import tpu as pltpu
from jax.experimental.pallas import tpu_sc as plsc

Bb, Ss, Dd = 4, 8192, 768
Ee, DFF, OUT = 64, 64, 768
Tt = Bb * Ss
CAP = 512
TB = 256
NBLK = Tt // TB
NOISE_STD = 1.0 / Ee
UROWS = Ee * CAP + Tt
NW = 32
TPW = Tt // NW
CHUNK = 64
NCH = TPW // CHUNK
NLEV = 6
ONE_BITS = 0x3F800000


def _router_body(x_ref, gw_ref, gb_ref, lns_ref, lnb_ref, nz_ref,
                 xn_ref, acc_ref):
    i = pl.program_id(0)

    @pl.when(i == 0)
    def _init():
        acc_ref[...] = jnp.zeros_like(acc_ref)

    xb = x_ref[...]
    mu = jnp.mean(xb, axis=-1, keepdims=True)
    xc = xb - mu
    var = jnp.mean(xc * xc, axis=-1, keepdims=True)
    xn = xc * lax.rsqrt(var + 1e-5) * lns_ref[...] + lnb_ref[...]
    xn_ref[...] = xn

    logits = jnp.dot(xn, gw_ref[...], preferred_element_type=jnp.float32)
    logits = logits + gb_ref[...]
    noisy = logits + nz_ref[...]
    nm = jnp.max(noisy, axis=-1, keepdims=True)

    m = jnp.max(logits, axis=-1, keepdims=True)
    eg = jnp.exp(logits - m)
    gates = eg / jnp.sum(eg, axis=-1, keepdims=True)
    acc_ref[0:1, 0:Ee] += jnp.sum(gates, axis=0, keepdims=True)

    z = (nm - logits) * (1.0 / NOISE_STD) * 0.7071067811865476
    p = 0.5 * (1.0 - lax.erf(z))
    acc_ref[1:2, 0:Ee] += jnp.sum(p, axis=0, keepdims=True)


NV = Ee * NLEV


def _plan_body(eid_ref, lev_ref, acc_ref, dst_ref, aux_ref, cnt_ref):
    p = pl.program_id(0)
    b = pl.program_id(1)
    e_row = eid_ref[0]
    l_row = lev_ref[0]
    comb = l_row * Ee + e_row
    iota_v = lax.broadcasted_iota(jnp.int32, (NV, 1), 0)
    oh = (iota_v == comb).astype(jnp.float32)
    colsum = jnp.sum(oh, axis=1, keepdims=True)

    @pl.when((p == 0) & (b == 0))
    def _zero():
        cnt_ref[...] = jnp.zeros_like(cnt_ref)

    @pl.when(p == 0)
    def _count():
        cnt_ref[:, 0:1] += colsum
        dst_ref[...] = jnp.zeros_like(dst_ref)

    @pl.when((p == 1) & (b == 0))
    def _mkbase():
        cnt = cnt_ref[:, 0:1]
        parts = [jnp.zeros((Ee, 1), jnp.float32)]
        for lv in range(1, NLEV):
            parts.append(parts[-1] + cnt[(lv - 1) * Ee:lv * Ee])
        cnt_ref[:, 0:1] = jnp.concatenate(parts, axis=0)

    @pl.when(p == 1)
    def _assign():
        rr = lax.broadcasted_iota(jnp.int32, (TB, TB), 0)
        cc = lax.broadcasted_iota(jnp.int32, (TB, TB), 1)
        tri = (rr < cc).astype(jnp.float32)
        cum = jnp.dot(oh, tri, preferred_element_type=jnp.float32)
        pos = cum + cnt_ref[:, 0:1]
        slot = jnp.sum(oh * pos, axis=0, keepdims=True)
        sloti = slot.astype(jnp.int32)
        tok = b * TB + lax.broadcasted_iota(jnp.int32, (1, TB), 1)
        dst_ref[0] = jnp.where(sloti < CAP, e_row * CAP + sloti,
                               Ee * CAP + tok)
        cnt_ref[:, 0:1] += colsum

    @pl.when((p == 1) & (b == NBLK - 1))
    def _aux():
        imp = acc_ref[0:1, 0:Ee]
        pm = acc_ref[1:2, 0:Ee] * (1.0 / Tt)
        im = jnp.sum(imp) * (1.0 / Ee)
        iv = jnp.sum((imp - im) ** 2) * (1.0 / (Ee - 1))
        pmm = jnp.sum(pm) * (1.0 / Ee)
        pv = jnp.sum((pm - pmm) ** 2) * (1.0 / (Ee - 1))
        imp_loss = iv / (im + 1e-8) ** 2
        load_loss = pv / (pmm + 1e-8) ** 2
        aux_ref[0, 0] = 0.5 * (imp_loss + load_loss)


def _ffn_body(u_ref, w1_ref, b1_ref, w2_ref, b2_ref, o_ref):
    xb = u_ref[...]
    h = jnp.dot(xb, w1_ref[0], preferred_element_type=jnp.float32) + b1_ref[0]
    h = jax.nn.gelu(h)
    o_ref[...] = jnp.dot(h, w2_ref[0], preferred_element_type=jnp.float32) + b2_ref[0]


def _dispatch_body(xn_hbm, dst_hbm, u_hbm, idx_v, buf_v, sem):
    wid = lax.axis_index("s") * 2 + lax.axis_index("c")
    base = wid * TPW
    pltpu.sync_copy(dst_hbm.at[wid], idx_v)
    for j in range(NCH):
        pltpu.sync_copy(xn_hbm.at[pl.ds(base + j * CHUNK, CHUNK)], buf_v)
        pltpu.async_copy(buf_v, u_hbm.at[idx_v.at[j]], sem).wait()


def _combine_body(u_hbm, dst_hbm, out_hbm, idx_v, buf_v, sem):
    wid = lax.axis_index("s") * 2 + lax.axis_index("c")
    base = wid * TPW
    pltpu.sync_copy(dst_hbm.at[wid], idx_v)
    for j in range(NCH):
        pltpu.async_copy(u_hbm.at[idx_v.at[j]], buf_v, sem).wait()
        pltpu.sync_copy(buf_v, out_hbm.at[pl.ds(base + j * CHUNK, CHUNK)])


def _run_router(xf, gate_W, gate_b, ln_scale, ln_bias, noise):
    return pl.pallas_call(
        _router_body,
        grid=(NBLK,),
        in_specs=[
            pl.BlockSpec((TB, Dd), lambda i: (i, 0)),
            pl.BlockSpec((Dd, Ee), lambda i: (0, 0)),
            pl.BlockSpec((1, Ee), lambda i: (0, 0)),
            pl.BlockSpec((1, Dd), lambda i: (0, 0)),
            pl.BlockSpec((1, Dd), lambda i: (0, 0)),
            pl.BlockSpec((TB, Ee), lambda i: (i, 0)),
        ],
        out_specs=[
            pl.BlockSpec((TB, Dd), lambda i: (i, 0)),
            pl.BlockSpec((8, 128), lambda i: (0, 0)),
        ],
        out_shape=[
            jax.ShapeDtypeStruct((Tt, Dd), jnp.float32),
            jax.ShapeDtypeStruct((8, 128), jnp.float32),
        ],
    )(xf, gate_W, gate_b.reshape(1, Ee), ln_scale.reshape(1, Dd),
      ln_bias.reshape(1, Dd), noise)


def _run_plan(eid, lev, acc):
    return pl.pallas_call(
        _plan_body,
        grid=(2, NBLK),
        in_specs=[
            pl.BlockSpec((1, 1, TB), lambda p, b: (b, 0, 0)),
            pl.BlockSpec((1, 1, TB), lambda p, b: (b, 0, 0)),
            pl.BlockSpec((8, 128), lambda p, b: (0, 0)),
        ],
        out_specs=[
            pl.BlockSpec((1, 1, TB), lambda p, b: (p * b, 0, 0)),
            pl.BlockSpec((1, 1), lambda p, b: (0, 0),
                         memory_space=pltpu.SMEM),
        ],
        out_shape=[
            jax.ShapeDtypeStruct((NBLK, 1, TB), jnp.int32),
            jax.ShapeDtypeStruct((1, 1), jnp.float32),
        ],
        scratch_shapes=[pltpu.VMEM((NV, 128), jnp.float32)],
    )(eid, lev, acc)


def kernel(x, gate_W, gate_b, ln_scale, ln_bias, W1, b1, W2, b2):
    xf = x.reshape(Tt, Dd)
    noise = NOISE_STD * jax.random.normal(
        jax.random.key(42), (Tt, Ee), dtype=jnp.float32)

    xn, acc = _run_router(xf, gate_W, gate_b, ln_scale, ln_bias, noise)

    mu = xf.mean(-1, keepdims=True)
    var = ((xf - mu) ** 2).mean(-1, keepdims=True)
    xr = (xf - mu) / jnp.sqrt(var + 1e-5) * ln_scale + ln_bias
    logits_r = xr @ gate_W + gate_b
    gates_noisy = jax.nn.softmax(logits_r + noise, axis=-1)
    topv, topi = jax.lax.top_k(gates_noisy, 1)
    wsel = topv / (topv.sum(-1, keepdims=True) + 1e-20)
    bits = jax.lax.bitcast_convert_type(wsel[:, 0], jnp.int32)
    lev = jnp.clip((ONE_BITS + 2) - bits, 0, NLEV - 1)
    eid = topi[:, 0].astype(jnp.int32).reshape(NBLK, 1, TB)
    lev = lev.reshape(NBLK, 1, TB)

    dst, aux = _run_plan(eid, lev, acc)

    dst_sc = dst.reshape(NW, NCH, CHUNK)
    mesh = plsc.VectorSubcoreMesh(core_axis_name="c", subcore_axis_name="s",
                                  num_cores=2, num_subcores=16)

    u0 = pl.kernel(
        _dispatch_body,
        out_type=jax.ShapeDtypeStruct((UROWS, Dd), jnp.float32),
        mesh=mesh,
        scratch_types=[
            pltpu.VMEM((NCH, CHUNK), jnp.int32),
            pltpu.VMEM((CHUNK, Dd), jnp.float32),
            pltpu.SemaphoreType.DMA,
        ],
    )(xn, dst_sc)

    u1 = pl.pallas_call(
        _ffn_body,
        grid=(Ee,),
        in_specs=[
            pl.BlockSpec((CAP, Dd), lambda e: (e, 0)),
            pl.BlockSpec((1, Dd, DFF), lambda e: (e, 0, 0)),
            pl.BlockSpec((1, 1, DFF), lambda e: (e, 0, 0)),
            pl.BlockSpec((1, DFF, OUT), lambda e: (e, 0, 0)),
            pl.BlockSpec((1, 1, OUT), lambda e: (e, 0, 0)),
        ],
        out_specs=pl.BlockSpec((CAP, Dd), lambda e: (e, 0)),
        out_shape=jax.ShapeDtypeStruct((UROWS, Dd), jnp.float32),
        input_output_aliases={0: 0},
    )(u0, W1, b1.reshape(Ee, 1, DFF), W2, b2.reshape(Ee, 1, OUT))

    out = pl.kernel(
        _combine_body,
        out_type=jax.ShapeDtypeStruct((Tt, Dd), jnp.float32),
        mesh=mesh,
        scratch_types=[
            pltpu.VMEM((NCH, CHUNK), jnp.int32),
            pltpu.VMEM((CHUNK, Dd), jnp.float32),
            pltpu.SemaphoreType.DMA,
        ],
    )(u1, dst_sc)

    return out.reshape(Bb, Ss, OUT), aux[0, 0]


# speedup vs baseline: 6.7284x; 6.7284x over previous
import functools

import jax
import jax.numpy as jnp
from jax import lax
from jax.experimental import pallas as pl
from jax.experimental.pallas import tpu as pltpu
from jax.experimental.pallas import tpu_sc as plsc

Bb, Ss, Dd = 4, 8192, 768
Ee, DFF, OUT = 64, 64, 768
Tt = Bb * Ss
CAP = 512
TB = 256
NBLK = Tt // TB
NOISE_STD = 1.0 / Ee
UROWS = Ee * CAP + Tt
NW = 32
TPW = Tt // NW
CHUNK = 64
NCH = TPW // CHUNK
NLEV = 6
ONE_BITS = 0x3F800000


def _router_body(x_ref, gw_ref, gb_ref, lns_ref, lnb_ref, nz_ref,
                 xn_ref, acc_ref):
    i = pl.program_id(0)

    @pl.when(i == 0)
    def _init():
        acc_ref[...] = jnp.zeros_like(acc_ref)

    xb = x_ref[...]
    mu = jnp.mean(xb, axis=-1, keepdims=True)
    xc = xb - mu
    var = jnp.mean(xc * xc, axis=-1, keepdims=True)
    xn = xc * lax.rsqrt(var + 1e-5) * lns_ref[...] + lnb_ref[...]
    xn_ref[...] = xn

    logits = jnp.dot(xn, gw_ref[...], preferred_element_type=jnp.float32)
    logits = logits + gb_ref[...]
    noisy = logits + nz_ref[...]
    nm = jnp.max(noisy, axis=-1, keepdims=True)

    m = jnp.max(logits, axis=-1, keepdims=True)
    eg = jnp.exp(logits - m)
    gates = eg / jnp.sum(eg, axis=-1, keepdims=True)
    acc_ref[0:1, 0:Ee] += jnp.sum(gates, axis=0, keepdims=True)

    z = (nm - logits) * (1.0 / NOISE_STD) * 0.7071067811865476
    p = 0.5 * (1.0 - lax.erf(z))
    acc_ref[1:2, 0:Ee] += jnp.sum(p, axis=0, keepdims=True)


NV = Ee * NLEV


def _plan_body(eid_ref, lev_ref, acc_ref, dst_ref, aux_ref, cnt_ref):
    p = pl.program_id(0)
    b = pl.program_id(1)
    e_row = eid_ref[0]
    l_row = lev_ref[0]
    comb = l_row * Ee + e_row
    iota_v = lax.broadcasted_iota(jnp.int32, (NV, 1), 0)
    oh = (iota_v == comb).astype(jnp.float32)
    colsum = jnp.sum(oh, axis=1, keepdims=True)

    @pl.when((p == 0) & (b == 0))
    def _zero():
        cnt_ref[...] = jnp.zeros_like(cnt_ref)

    @pl.when(p == 0)
    def _count():
        cnt_ref[:, 0:1] += colsum
        dst_ref[...] = jnp.zeros_like(dst_ref)

    @pl.when((p == 1) & (b == 0))
    def _mkbase():
        cnt = cnt_ref[:, 0:1]
        parts = [jnp.zeros((Ee, 1), jnp.float32)]
        for lv in range(1, NLEV):
            parts.append(parts[-1] + cnt[(lv - 1) * Ee:lv * Ee])
        cnt_ref[:, 0:1] = jnp.concatenate(parts, axis=0)

    @pl.when(p == 1)
    def _assign():
        rr = lax.broadcasted_iota(jnp.int32, (TB, TB), 0)
        cc = lax.broadcasted_iota(jnp.int32, (TB, TB), 1)
        tri = (rr < cc).astype(jnp.float32)
        cum = jnp.dot(oh, tri, preferred_element_type=jnp.float32)
        pos = cum + cnt_ref[:, 0:1]
        slot = jnp.sum(oh * pos, axis=0, keepdims=True)
        sloti = slot.astype(jnp.int32)
        tok = b * TB + lax.broadcasted_iota(jnp.int32, (1, TB), 1)
        dst_ref[0] = jnp.where(sloti < CAP, e_row * CAP + sloti,
                               Ee * CAP + tok)
        cnt_ref[:, 0:1] += colsum

    @pl.when((p == 1) & (b == NBLK - 1))
    def _aux():
        imp = acc_ref[0:1, 0:Ee]
        pm = acc_ref[1:2, 0:Ee] * (1.0 / Tt)
        im = jnp.sum(imp) * (1.0 / Ee)
        iv = jnp.sum((imp - im) ** 2) * (1.0 / (Ee - 1))
        pmm = jnp.sum(pm) * (1.0 / Ee)
        pv = jnp.sum((pm - pmm) ** 2) * (1.0 / (Ee - 1))
        imp_loss = iv / (im + 1e-8) ** 2
        load_loss = pv / (pmm + 1e-8) ** 2
        aux_ref[0, 0] = 0.5 * (imp_loss + load_loss)


def _ffn_body(u_ref, w1_ref, b1_ref, w2_ref, b2_ref, o_ref):
    xb = u_ref[...]
    h = jnp.dot(xb, w1_ref[0], preferred_element_type=jnp.float32) + b1_ref[0]
    h = jax.nn.gelu(h)
    o_ref[...] = jnp.dot(h, w2_ref[0], preferred_element_type=jnp.float32) + b2_ref[0]


def _dispatch_body(xn_hbm, dst_hbm, u_hbm, idx_v, buf_v, sem):
    wid = lax.axis_index("s") * 2 + lax.axis_index("c")
    base = wid * TPW
    pltpu.sync_copy(dst_hbm.at[wid], idx_v)
    for j in range(NCH):
        pltpu.sync_copy(xn_hbm.at[pl.ds(base + j * CHUNK, CHUNK)], buf_v)
        pltpu.async_copy(buf_v, u_hbm.at[idx_v.at[j]], sem).wait()


def _combine_body(u_hbm, dst_hbm, out_hbm, idx_v, buf_v, sem):
    wid = lax.axis_index("s") * 2 + lax.axis_index("c")
    base = wid * TPW
    pltpu.sync_copy(dst_hbm.at[wid], idx_v)
    for j in range(NCH):
        pltpu.async_copy(u_hbm.at[idx_v.at[j]], buf_v, sem).wait()
        pltpu.sync_copy(buf_v, out_hbm.at[pl.ds(base + j * CHUNK, CHUNK)])


def _run_router(xf, gate_W, gate_b, ln_scale, ln_bias, noise):
    return pl.pallas_call(
        _router_body,
        grid=(NBLK,),
        in_specs=[
            pl.BlockSpec((TB, Dd), lambda i: (i, 0)),
            pl.BlockSpec((Dd, Ee), lambda i: (0, 0)),
            pl.BlockSpec((1, Ee), lambda i: (0, 0)),
            pl.BlockSpec((1, Dd), lambda i: (0, 0)),
            pl.BlockSpec((1, Dd), lambda i: (0, 0)),
            pl.BlockSpec((TB, Ee), lambda i: (i, 0)),
        ],
        out_specs=[
            pl.BlockSpec((TB, Dd), lambda i: (i, 0)),
            pl.BlockSpec((8, 128), lambda i: (0, 0)),
        ],
        out_shape=[
            jax.ShapeDtypeStruct((Tt, Dd), jnp.float32),
            jax.ShapeDtypeStruct((8, 128), jnp.float32),
        ],
    )(xf, gate_W, gate_b.reshape(1, Ee), ln_scale.reshape(1, Dd),
      ln_bias.reshape(1, Dd), noise)


def _run_plan(eid, lev, acc):
    return pl.pallas_call(
        _plan_body,
        grid=(2, NBLK),
        in_specs=[
            pl.BlockSpec((1, 1, TB), lambda p, b: (b, 0, 0)),
            pl.BlockSpec((1, 1, TB), lambda p, b: (b, 0, 0)),
            pl.BlockSpec((8, 128), lambda p, b: (0, 0)),
        ],
        out_specs=[
            pl.BlockSpec((1, 1, TB), lambda p, b: (p * b, 0, 0)),
            pl.BlockSpec((1, 1), lambda p, b: (0, 0),
                         memory_space=pltpu.SMEM),
        ],
        out_shape=[
            jax.ShapeDtypeStruct((NBLK, 1, TB), jnp.int32),
            jax.ShapeDtypeStruct((1, 1), jnp.float32),
        ],
        scratch_shapes=[pltpu.VMEM((NV, 128), jnp.float32)],
    )(eid, lev, acc)


def kernel(x, gate_W, gate_b, ln_scale, ln_bias, W1, b1, W2, b2):
    xf = x.reshape(Tt, Dd)
    noise = NOISE_STD * jax.random.normal(
        jax.random.key(42), (Tt, Ee), dtype=jnp.float32)

    xn, acc = _run_router(xf, gate_W, gate_b, ln_scale, ln_bias, noise)

    mu = xf.mean(-1, keepdims=True)
    var = ((xf - mu) ** 2).mean(-1, keepdims=True)
    xr = (xf - mu) / jnp.sqrt(var + 1e-5) * ln_scale + ln_bias
    logits_r = xr @ gate_W + gate_b
    gates_noisy = jax.nn.softmax(logits_r + noise, axis=-1)
    topv = jnp.max(gates_noisy, axis=-1, keepdims=True)
    topi = jnp.argmax(gates_noisy, axis=-1)
    wsel = topv / (topv.sum(-1, keepdims=True) + 1e-20)
    bits = jax.lax.bitcast_convert_type(wsel[:, 0], jnp.int32)
    lev = jnp.clip((ONE_BITS + 2) - bits, 0, NLEV - 1)
    eid = topi.astype(jnp.int32).reshape(NBLK, 1, TB)
    lev = lev.reshape(NBLK, 1, TB)

    dst, aux = _run_plan(eid, lev, acc)

    dst_sc = dst.reshape(NW, NCH, CHUNK)
    mesh = plsc.VectorSubcoreMesh(core_axis_name="c", subcore_axis_name="s",
                                  num_cores=2, num_subcores=16)

    u0 = pl.kernel(
        _dispatch_body,
        out_type=jax.ShapeDtypeStruct((UROWS, Dd), jnp.float32),
        mesh=mesh,
        scratch_types=[
            pltpu.VMEM((NCH, CHUNK), jnp.int32),
            pltpu.VMEM((CHUNK, Dd), jnp.float32),
            pltpu.SemaphoreType.DMA,
        ],
    )(xn, dst_sc)

    u1 = pl.pallas_call(
        _ffn_body,
        grid=(Ee,),
        in_specs=[
            pl.BlockSpec((CAP, Dd), lambda e: (e, 0)),
            pl.BlockSpec((1, Dd, DFF), lambda e: (e, 0, 0)),
            pl.BlockSpec((1, 1, DFF), lambda e: (e, 0, 0)),
            pl.BlockSpec((1, DFF, OUT), lambda e: (e, 0, 0)),
            pl.BlockSpec((1, 1, OUT), lambda e: (e, 0, 0)),
        ],
        out_specs=pl.BlockSpec((CAP, Dd), lambda e: (e, 0)),
        out_shape=jax.ShapeDtypeStruct((UROWS, Dd), jnp.float32),
        input_output_aliases={0: 0},
    )(u0, W1, b1.reshape(Ee, 1, DFF), W2, b2.reshape(Ee, 1, OUT))

    out = pl.kernel(
        _combine_body,
        out_type=jax.ShapeDtypeStruct((Tt, Dd), jnp.float32),
        mesh=mesh,
        scratch_types=[
            pltpu.VMEM((NCH, CHUNK), jnp.int32),
            pltpu.VMEM((CHUNK, Dd), jnp.float32),
            pltpu.SemaphoreType.DMA,
        ],
    )(u1, dst_sc)

    return out.reshape(Bb, Ss, OUT), aux[0, 0]


# speedup vs baseline: 6.8779x; 1.0222x over previous
import functools

import jax
import jax.numpy as jnp
from jax import lax
from jax.experimental import pallas as pl
from jax.experimental.pallas import tpu as pltpu
from jax.experimental.pallas import tpu_sc as plsc

Bb, Ss, Dd = 4, 8192, 768
Ee, DFF, OUT = 64, 64, 768
Tt = Bb * Ss
CAP = 512
TB = 256
NBLK = Tt // TB
NOISE_STD = 1.0 / Ee
UROWS = Ee * CAP + Tt
NW = 32
TPW = Tt // NW
CHUNK = 64
NCH = TPW // CHUNK
NLEV = 6
ONE_BITS = 0x3F800000


def _router_body(x_ref, gw_ref, gb_ref, lns_ref, lnb_ref, nz_ref,
                 xn_ref, acc_ref):
    i = pl.program_id(0)

    @pl.when(i == 0)
    def _init():
        acc_ref[...] = jnp.zeros_like(acc_ref)

    xb = x_ref[...]
    mu = jnp.mean(xb, axis=-1, keepdims=True)
    xc = xb - mu
    var = jnp.mean(xc * xc, axis=-1, keepdims=True)
    xn = xc * lax.rsqrt(var + 1e-5) * lns_ref[...] + lnb_ref[...]
    xn_ref[...] = xn

    logits = jnp.dot(xn, gw_ref[...], preferred_element_type=jnp.float32)
    logits = logits + gb_ref[...]
    noisy = logits + nz_ref[...]
    nm = jnp.max(noisy, axis=-1, keepdims=True)

    m = jnp.max(logits, axis=-1, keepdims=True)
    eg = jnp.exp(logits - m)
    gates = eg / jnp.sum(eg, axis=-1, keepdims=True)
    acc_ref[0:1, 0:Ee] += jnp.sum(gates, axis=0, keepdims=True)

    z = (nm - logits) * (1.0 / NOISE_STD) * 0.7071067811865476
    p = 0.5 * (1.0 - lax.erf(z))
    acc_ref[1:2, 0:Ee] += jnp.sum(p, axis=0, keepdims=True)


NV = Ee * NLEV


def _plan_body(eid_ref, lev_ref, acc_ref, dst_ref, aux_ref, cnt_ref):
    p = pl.program_id(0)
    b = pl.program_id(1)
    e_row = eid_ref[0]
    l_row = lev_ref[0]
    comb = l_row * Ee + e_row
    iota_v = lax.broadcasted_iota(jnp.int32, (NV, 1), 0)
    oh = (iota_v == comb).astype(jnp.float32)
    colsum = jnp.sum(oh, axis=1, keepdims=True)

    @pl.when((p == 0) & (b == 0))
    def _zero():
        cnt_ref[...] = jnp.zeros_like(cnt_ref)

    @pl.when(p == 0)
    def _count():
        cnt_ref[:, 0:1] += colsum
        dst_ref[...] = jnp.zeros_like(dst_ref)

    @pl.when((p == 1) & (b == 0))
    def _mkbase():
        cnt = cnt_ref[:, 0:1]
        parts = [jnp.zeros((Ee, 1), jnp.float32)]
        for lv in range(1, NLEV):
            parts.append(parts[-1] + cnt[(lv - 1) * Ee:lv * Ee])
        cnt_ref[:, 0:1] = jnp.concatenate(parts, axis=0)

    @pl.when(p == 1)
    def _assign():
        rr = lax.broadcasted_iota(jnp.int32, (TB, TB), 0)
        cc = lax.broadcasted_iota(jnp.int32, (TB, TB), 1)
        tri = (rr < cc).astype(jnp.float32)
        cum = jnp.dot(oh, tri, preferred_element_type=jnp.float32)
        pos = cum + cnt_ref[:, 0:1]
        slot = jnp.sum(oh * pos, axis=0, keepdims=True)
        sloti = slot.astype(jnp.int32)
        tok = b * TB + lax.broadcasted_iota(jnp.int32, (1, TB), 1)
        dst_ref[0] = jnp.where(sloti < CAP, e_row * CAP + sloti,
                               Ee * CAP + tok)
        cnt_ref[:, 0:1] += colsum

    @pl.when((p == 1) & (b == NBLK - 1))
    def _aux():
        imp = acc_ref[0:1, 0:Ee]
        pm = acc_ref[1:2, 0:Ee] * (1.0 / Tt)
        im = jnp.sum(imp) * (1.0 / Ee)
        iv = jnp.sum((imp - im) ** 2) * (1.0 / (Ee - 1))
        pmm = jnp.sum(pm) * (1.0 / Ee)
        pv = jnp.sum((pm - pmm) ** 2) * (1.0 / (Ee - 1))
        imp_loss = iv / (im + 1e-8) ** 2
        load_loss = pv / (pmm + 1e-8) ** 2
        aux_ref[0, 0] = 0.5 * (imp_loss + load_loss)


def _ffn_body(u_ref, w1_ref, b1_ref, w2_ref, b2_ref, o_ref):
    xb = u_ref[...]
    h = jnp.dot(xb, w1_ref[0], preferred_element_type=jnp.float32) + b1_ref[0]
    h = jax.nn.gelu(h)
    o_ref[...] = jnp.dot(h, w2_ref[0], preferred_element_type=jnp.float32) + b2_ref[0]


def _dispatch_body(xn_hbm, dst_hbm, u_hbm, idx_v, buf0, buf1, sem0, sem1, semsc):
    wid = lax.axis_index("s") * 2 + lax.axis_index("c")
    base = wid * TPW
    pltpu.sync_copy(dst_hbm.at[wid], idx_v)
    bufs, sems = (buf0, buf1), (sem0, sem1)
    pend = [None, None]
    pend[0] = pltpu.async_copy(
        xn_hbm.at[pl.ds(base, CHUNK)], buf0, sem0)
    for j in range(NCH):
        if j + 1 < NCH:
            pend[(j + 1) % 2] = pltpu.async_copy(
                xn_hbm.at[pl.ds(base + (j + 1) * CHUNK, CHUNK)],
                bufs[(j + 1) % 2], sems[(j + 1) % 2])
        pend[j % 2].wait()
        pltpu.async_copy(bufs[j % 2], u_hbm.at[idx_v.at[j]], semsc).wait()


def _combine_body(u_hbm, dst_hbm, out_hbm, idx_v, buf0, buf1, sem0, sem1):
    wid = lax.axis_index("s") * 2 + lax.axis_index("c")
    base = wid * TPW
    pltpu.sync_copy(dst_hbm.at[wid], idx_v)
    bufs, sems = (buf0, buf1), (sem0, sem1)
    pend = [None, None]
    pend[0] = pltpu.async_copy(u_hbm.at[idx_v.at[0]], buf0, sem0)
    for j in range(NCH):
        if j + 1 < NCH:
            pend[(j + 1) % 2] = pltpu.async_copy(
                u_hbm.at[idx_v.at[j + 1]], bufs[(j + 1) % 2],
                sems[(j + 1) % 2])
        pend[j % 2].wait()
        pltpu.sync_copy(bufs[j % 2],
                        out_hbm.at[pl.ds(base + j * CHUNK, CHUNK)])


def _run_router(xf, gate_W, gate_b, ln_scale, ln_bias, noise):
    return pl.pallas_call(
        _router_body,
        grid=(NBLK,),
        in_specs=[
            pl.BlockSpec((TB, Dd), lambda i: (i, 0)),
            pl.BlockSpec((Dd, Ee), lambda i: (0, 0)),
            pl.BlockSpec((1, Ee), lambda i: (0, 0)),
            pl.BlockSpec((1, Dd), lambda i: (0, 0)),
            pl.BlockSpec((1, Dd), lambda i: (0, 0)),
            pl.BlockSpec((TB, Ee), lambda i: (i, 0)),
        ],
        out_specs=[
            pl.BlockSpec((TB, Dd), lambda i: (i, 0)),
            pl.BlockSpec((8, 128), lambda i: (0, 0)),
        ],
        out_shape=[
            jax.ShapeDtypeStruct((Tt, Dd), jnp.float32),
            jax.ShapeDtypeStruct((8, 128), jnp.float32),
        ],
    )(xf, gate_W, gate_b.reshape(1, Ee), ln_scale.reshape(1, Dd),
      ln_bias.reshape(1, Dd), noise)


def _run_plan(eid, lev, acc):
    return pl.pallas_call(
        _plan_body,
        grid=(2, NBLK),
        in_specs=[
            pl.BlockSpec((1, 1, TB), lambda p, b: (b, 0, 0)),
            pl.BlockSpec((1, 1, TB), lambda p, b: (b, 0, 0)),
            pl.BlockSpec((8, 128), lambda p, b: (0, 0)),
        ],
        out_specs=[
            pl.BlockSpec((1, 1, TB), lambda p, b: (p * b, 0, 0)),
            pl.BlockSpec((1, 1), lambda p, b: (0, 0),
                         memory_space=pltpu.SMEM),
        ],
        out_shape=[
            jax.ShapeDtypeStruct((NBLK, 1, TB), jnp.int32),
            jax.ShapeDtypeStruct((1, 1), jnp.float32),
        ],
        scratch_shapes=[pltpu.VMEM((NV, 128), jnp.float32)],
    )(eid, lev, acc)


def kernel(x, gate_W, gate_b, ln_scale, ln_bias, W1, b1, W2, b2):
    xf = x.reshape(Tt, Dd)
    noise = NOISE_STD * jax.random.normal(
        jax.random.key(42), (Tt, Ee), dtype=jnp.float32)

    xn, acc = _run_router(xf, gate_W, gate_b, ln_scale, ln_bias, noise)

    mu = xf.mean(-1, keepdims=True)
    var = ((xf - mu) ** 2).mean(-1, keepdims=True)
    xr = (xf - mu) / jnp.sqrt(var + 1e-5) * ln_scale + ln_bias
    logits_r = xr @ gate_W + gate_b
    gates_noisy = jax.nn.softmax(logits_r + noise, axis=-1)
    topv = jnp.max(gates_noisy, axis=-1, keepdims=True)
    topi = jnp.argmax(gates_noisy, axis=-1)
    wsel = topv / (topv.sum(-1, keepdims=True) + 1e-20)
    bits = jax.lax.bitcast_convert_type(wsel[:, 0], jnp.int32)
    lev = jnp.clip((ONE_BITS + 2) - bits, 0, NLEV - 1)
    eid = topi.astype(jnp.int32).reshape(NBLK, 1, TB)
    lev = lev.reshape(NBLK, 1, TB)

    dst, aux = _run_plan(eid, lev, acc)

    dst_sc = dst.reshape(NW, NCH, CHUNK)
    mesh = plsc.VectorSubcoreMesh(core_axis_name="c", subcore_axis_name="s",
                                  num_cores=2, num_subcores=16)

    u0 = pl.kernel(
        _dispatch_body,
        out_type=jax.ShapeDtypeStruct((UROWS, Dd), jnp.float32),
        mesh=mesh,
        scratch_types=[
            pltpu.VMEM((NCH, CHUNK), jnp.int32),
            pltpu.VMEM((CHUNK, Dd), jnp.float32),
            pltpu.VMEM((CHUNK, Dd), jnp.float32),
            pltpu.SemaphoreType.DMA,
            pltpu.SemaphoreType.DMA,
            pltpu.SemaphoreType.DMA,
        ],
    )(xn, dst_sc)

    u1 = pl.pallas_call(
        _ffn_body,
        grid=(Ee,),
        in_specs=[
            pl.BlockSpec((CAP, Dd), lambda e: (e, 0)),
            pl.BlockSpec((1, Dd, DFF), lambda e: (e, 0, 0)),
            pl.BlockSpec((1, 1, DFF), lambda e: (e, 0, 0)),
            pl.BlockSpec((1, DFF, OUT), lambda e: (e, 0, 0)),
            pl.BlockSpec((1, 1, OUT), lambda e: (e, 0, 0)),
        ],
        out_specs=pl.BlockSpec((CAP, Dd), lambda e: (e, 0)),
        out_shape=jax.ShapeDtypeStruct((UROWS, Dd), jnp.float32),
        input_output_aliases={0: 0},
    )(u0, W1, b1.reshape(Ee, 1, DFF), W2, b2.reshape(Ee, 1, OUT))

    out = pl.kernel(
        _combine_body,
        out_type=jax.ShapeDtypeStruct((Tt, Dd), jnp.float32),
        mesh=mesh,
        scratch_types=[
            pltpu.VMEM((NCH, CHUNK), jnp.int32),
            pltpu.VMEM((CHUNK, Dd), jnp.float32),
            pltpu.VMEM((CHUNK, Dd), jnp.float32),
            pltpu.SemaphoreType.DMA,
            pltpu.SemaphoreType.DMA,
        ],
    )(u1, dst_sc)

    return out.reshape(Bb, Ss, OUT), aux[0, 0]
